# baseline (speedup 1.0000x reference)
import functools

import jax
import jax.numpy as jnp
from jax import lax
from jax.experimental import pallas as pl
from jax.experimental.pallas import tpu as pltpu
from jax.experimental.pallas import tpu_sc as plsc

_BATCH = 4
_SEQ = 2048
_EMBED = 64
_VOCAB = 100000
_FLAT = _BATCH * _SEQ

_INFO = plsc.get_sparse_core_info()
_NC = _INFO.num_cores
_NS = _INFO.num_subcores
_NW = _NC * _NS
_ROWS_W = _FLAT // _NW
_LANES = 16
_TCHUNK = 32
_NCHUNK = _ROWS_W // _TCHUNK
_IDXROW = 128


def _round_bf16(s):
    u = plsc.bitcast(s, jnp.int32)
    lsb = lax.bitwise_and(lax.shift_right_logical(u, 16), 1)
    return u + 0x7FFF + lsb


def _emb_body(tok_hbm, table_hbm, pos_hbm, out_hbm, idx_v, dst0, dst1,
              prow_v, out_v, sem0, sem1):
    wid = lax.axis_index("s") * _NC + lax.axis_index("c")
    base = wid * _ROWS_W
    pltpu.sync_copy(tok_hbm, idx_v)
    pbase = pl.multiple_of(lax.rem(base, _SEQ), _ROWS_W)
    pltpu.sync_copy(pos_hbm.at[pl.ds(pbase, _ROWS_W)], prow_v)

    dsts = (dst0, dst1)
    sems = (sem0, sem1)
    iota = lax.iota(jnp.int32, _LANES)

    def tok_vec(gg):
        flat = base + gg * _LANES
        row = jnp.full((_LANES,), flat // _IDXROW, jnp.int32)
        col = iota + (flat % _IDXROW)
        return plsc.load_gather(idx_v, [row, col])

    def start_gather(q):
        cps = []
        for h in range(_TCHUNK // _LANES):
            pid = lax.shift_right_logical(tok_vec(q * 2 + h), 1)
            cps.append(pltpu.async_copy(
                table_hbm.at[pid],
                dsts[q % 2].at[pl.ds(h * _LANES, _LANES)], sems[q % 2]))
        return cps

    cps = start_gather(0)
    for q in range(_NCHUNK):
        for cp in cps:
            cp.wait()
        if q + 1 < _NCHUNK:
            cps = start_gather(q + 1)
        dst_b = dsts[q % 2]
        for h in range(_TCHUNK // _LANES):
            tokv = tok_vec(q * 2 + h)
            colbase = lax.bitwise_and(tokv, 1) * _EMBED
            t_loc = h * _LANES + iota
            item = t_loc
            t_wrk = q * _TCHUNK + t_loc

            def pair(cp_i, carry):
                c = cp_i * 2
                csp = jnp.full((_LANES,), 1, jnp.int32) * c
                a = plsc.load_gather(dst_b, [item, colbase + c])
                b = plsc.load_gather(dst_b, [item, colbase + (c + 1)])
                pa = plsc.load_gather(prow_v, [t_wrk, csp])
                pb = plsc.load_gather(prow_v, [t_wrk, csp + 1])
                ua = _round_bf16(a + pa)
                ub = _round_bf16(b + pb)
                w = lax.bitwise_or(lax.shift_right_logical(ua, 16),
                                   lax.bitwise_and(ub, jnp.int32(-65536)))
                wrow = jnp.full((_LANES,), 1, jnp.int32) * cp_i
                plsc.store_scatter(out_v, [wrow, t_wrk], w)
                return carry

            lax.fori_loop(0, _EMBED // 2, pair, 0)
    bidx = base // _SEQ
    s0 = pl.multiple_of(lax.rem(base, _SEQ), _ROWS_W)
    pltpu.sync_copy(out_v, out_hbm.at[bidx, :, pl.ds(s0, _ROWS_W)])


_emb = functools.partial(
    pl.kernel,
    mesh=plsc.VectorSubcoreMesh(core_axis_name="c", subcore_axis_name="s"),
    out_type=jax.ShapeDtypeStruct((_BATCH, _EMBED // 2, _SEQ), jnp.int32),
    scratch_types=[
        pltpu.VMEM((_FLAT // _IDXROW, _IDXROW), jnp.int32),
        pltpu.VMEM((_TCHUNK, 2 * _EMBED), jnp.float32),
        pltpu.VMEM((_TCHUNK, 2 * _EMBED), jnp.float32),
        pltpu.VMEM((_ROWS_W, _EMBED), jnp.float32),
        pltpu.VMEM((_EMBED // 2, _ROWS_W), jnp.int32),
        pltpu.SemaphoreType.DMA,
        pltpu.SemaphoreType.DMA,
    ],
    compiler_params=pltpu.CompilerParams(use_tc_tiling_on_sc=True,
                                         needs_layout_passes=False),
)(_emb_body)


def kernel(tokens, token_table, pos_table):
    tok = tokens.astype(jnp.int32).reshape(_FLAT // _IDXROW, _IDXROW)
    table2 = token_table.reshape(_VOCAB // 2, 2 * _EMBED)
    out = _emb(tok, table2, pos_table)
    pairs = lax.bitcast_convert_type(out, jnp.bfloat16)
    return jnp.transpose(pairs, (0, 2, 1, 3)).reshape(_BATCH, _SEQ, _EMBED)


# speedup vs baseline: 1.0057x; 1.0057x over previous
import functools

import jax
import jax.numpy as jnp
from jax import lax
from jax.experimental import pallas as pl
from jax.experimental.pallas import tpu as pltpu
from jax.experimental.pallas import tpu_sc as plsc

_BATCH = 4
_SEQ = 2048
_EMBED = 64
_VOCAB = 100000
_FLAT = _BATCH * _SEQ

_INFO = plsc.get_sparse_core_info()
_NC = _INFO.num_cores
_NS = _INFO.num_subcores
_NW = _NC * _NS
_ROWS_W = _FLAT // _NW
_LANES = 16
_TCHUNK = 32
_NCHUNK = _ROWS_W // _TCHUNK
_IDXROW = 128


def _round_bf16(s):
    u = plsc.bitcast(s, jnp.int32)
    lsb = lax.bitwise_and(lax.shift_right_logical(u, 16), 1)
    return u + 0x7FFF + lsb


def _emb_body(tok_hbm, table_hbm, pos_hbm, out_hbm, idx_v, dst_v, prow_v,
              out_v, sem):
    wid = lax.axis_index("s") * _NC + lax.axis_index("c")
    base = wid * _ROWS_W
    pltpu.sync_copy(tok_hbm, idx_v)
    pbase = pl.multiple_of(lax.rem(base, _SEQ), _ROWS_W)
    pltpu.sync_copy(pos_hbm.at[pl.ds(pbase, _ROWS_W)], prow_v)

    iota = lax.iota(jnp.int32, _LANES)

    def tok_vec(gg):
        flat = base + gg * _LANES
        row = jnp.full((_LANES,), 1, jnp.int32) * (flat // _IDXROW)
        col = iota + lax.rem(flat, _IDXROW)
        return plsc.load_gather(idx_v, [row, col])

    cps = []
    for g in range(_ROWS_W // _LANES):
        pid = lax.shift_right_logical(tok_vec(g), 1)
        cps.append(pltpu.async_copy(
            table_hbm.at[pid], dst_v.at[pl.ds(g * _LANES, _LANES)], sem))
    for cp in cps:
        cp.wait()

    def group(g, carry):
        tokv = tok_vec(g)
        colbase = lax.bitwise_and(tokv, 1) * _EMBED
        t_wrk = g * _LANES + iota
        for cp_i in range(_EMBED // 2):
            c = cp_i * 2
            csp = jnp.full((_LANES,), 1, jnp.int32) * c
            a = plsc.load_gather(dst_v, [t_wrk, colbase + c])
            b = plsc.load_gather(dst_v, [t_wrk, colbase + (c + 1)])
            pa = plsc.load_gather(prow_v, [t_wrk, csp])
            pb = plsc.load_gather(prow_v, [t_wrk, csp + 1])
            ua = _round_bf16(a + pa)
            ub = _round_bf16(b + pb)
            w = lax.bitwise_or(lax.shift_right_logical(ua, 16),
                               lax.bitwise_and(ub, jnp.int32(-65536)))
            wrow = jnp.full((_LANES,), 1, jnp.int32) * cp_i
            plsc.store_scatter(out_v, [wrow, t_wrk], w)
        return carry

    lax.fori_loop(0, _ROWS_W // _LANES, group, 0)
    bidx = base // _SEQ
    pltpu.sync_copy(out_v, out_hbm.at[bidx, :, pl.ds(pbase, _ROWS_W)])


_emb = functools.partial(
    pl.kernel,
    mesh=plsc.VectorSubcoreMesh(core_axis_name="c", subcore_axis_name="s"),
    out_type=jax.ShapeDtypeStruct((_BATCH, _EMBED // 2, _SEQ), jnp.int32),
    scratch_types=[
        pltpu.VMEM((_FLAT // _IDXROW, _IDXROW), jnp.int32),
        pltpu.VMEM((_ROWS_W, 2 * _EMBED), jnp.float32),
        pltpu.VMEM((_ROWS_W, _EMBED), jnp.float32),
        pltpu.VMEM((_EMBED // 2, _ROWS_W), jnp.int32),
        pltpu.SemaphoreType.DMA,
    ],
    compiler_params=pltpu.CompilerParams(use_tc_tiling_on_sc=True,
                                         needs_layout_passes=False),
)(_emb_body)


def kernel(tokens, token_table, pos_table):
    tok = tokens.astype(jnp.int32).reshape(_FLAT // _IDXROW, _IDXROW)
    table2 = token_table.reshape(_VOCAB // 2, 2 * _EMBED)
    out = _emb(tok, table2, pos_table)
    pairs = lax.bitcast_convert_type(out, jnp.bfloat16)
    return jnp.transpose(pairs, (0, 2, 1, 3)).reshape(_BATCH, _SEQ, _EMBED)


# speedup vs baseline: 1.0153x; 1.0096x over previous
import functools

import jax
import jax.numpy as jnp
from jax import lax
from jax.experimental import pallas as pl
from jax.experimental.pallas import tpu as pltpu
from jax.experimental.pallas import tpu_sc as plsc

_BATCH = 4
_SEQ = 2048
_EMBED = 64
_VOCAB = 100000
_FLAT = _BATCH * _SEQ

_INFO = plsc.get_sparse_core_info()
_NC = _INFO.num_cores
_NS = _INFO.num_subcores
_NW = _NC * _NS
_ROWS_W = _FLAT // _NW
_LANES = 16
_TCHUNK = 32
_NCHUNK = _ROWS_W // _TCHUNK
_IDXROW = 128


def _round_bf16(s):
    u = plsc.bitcast(s, jnp.int32)
    lsb = lax.bitwise_and(lax.shift_right_logical(u, 16), 1)
    return u + 0x7FFF + lsb


def _emb_body(tok_hbm, table_hbm, pos_hbm, out_hbm, idx_v, dst_v, prow_v,
              out_v, sem):
    wid = lax.axis_index("s") * _NC + lax.axis_index("c")
    base = wid * _ROWS_W
    pltpu.sync_copy(tok_hbm, idx_v)
    pbase = pl.multiple_of(lax.rem(base, _SEQ), _ROWS_W)
    pltpu.sync_copy(pos_hbm.at[pl.ds(pbase, _ROWS_W)], prow_v)

    iota = lax.iota(jnp.int32, _LANES)

    def tok_vec(gg):
        flat = base + gg * _LANES
        row = jnp.full((_LANES,), 1, jnp.int32) * (flat // _IDXROW)
        col = iota + lax.rem(flat, _IDXROW)
        return plsc.load_gather(idx_v, [row, col])

    cps = []
    for g in range(_ROWS_W // _LANES):
        pid = lax.shift_right_logical(tok_vec(g), 1)
        cps.append(pltpu.async_copy(
            table_hbm.at[pid], dst_v.at[pl.ds(g * _LANES, _LANES)], sem))
    for cp in cps:
        cp.wait()

    def group(g, carry):
        tokv = tok_vec(g)
        colbase = lax.bitwise_and(tokv, 1) * _EMBED
        t_wrk = g * _LANES + iota
        for cp_i in range(_EMBED // 2):
            c = cp_i * 2
            csp = jnp.full((_LANES,), 1, jnp.int32) * c
            a = plsc.load_gather(dst_v, [t_wrk, colbase + c])
            b = plsc.load_gather(dst_v, [t_wrk, colbase + (c + 1)])
            pa = plsc.load_gather(prow_v, [t_wrk, csp])
            pb = plsc.load_gather(prow_v, [t_wrk, csp + 1])
            ua = _round_bf16(a + pa)
            ub = _round_bf16(b + pb)
            w = lax.bitwise_or(lax.shift_right_logical(ua, 16),
                               lax.bitwise_and(ub, jnp.int32(-65536)))
            wrow = jnp.full((_LANES,), 1, jnp.int32) * cp_i
            plsc.store_scatter(out_v, [wrow, t_wrk], w)
        return carry

    lax.fori_loop(0, _ROWS_W // _LANES, group, 0)
    bidx = base // _SEQ
    pltpu.sync_copy(out_v, out_hbm.at[bidx, :, pl.ds(pbase, _ROWS_W)])


_emb = functools.partial(
    pl.kernel,
    mesh=plsc.VectorSubcoreMesh(core_axis_name="c", subcore_axis_name="s"),
    out_type=jax.ShapeDtypeStruct((_BATCH, _EMBED // 2, _SEQ), jnp.int32),
    scratch_types=[
        pltpu.VMEM((_FLAT // _IDXROW, _IDXROW), jnp.int32),
        pltpu.VMEM((_ROWS_W, 2 * _EMBED), jnp.float32),
        pltpu.VMEM((_ROWS_W, _EMBED), jnp.float32),
        pltpu.VMEM((_EMBED // 2, _ROWS_W), jnp.int32),
        pltpu.SemaphoreType.DMA,
    ],
    compiler_params=pltpu.CompilerParams(use_tc_tiling_on_sc=False,
                                         needs_layout_passes=False),
)(_emb_body)


def kernel(tokens, token_table, pos_table):
    tok = tokens.astype(jnp.int32).reshape(_FLAT // _IDXROW, _IDXROW)
    table2 = token_table.reshape(_VOCAB // 2, 2 * _EMBED)
    out = _emb(tok, table2, pos_table)
    pairs = lax.bitcast_convert_type(out, jnp.bfloat16)
    return jnp.transpose(pairs, (0, 2, 1, 3)).reshape(_BATCH, _SEQ, _EMBED)
